# baseline (speedup 1.0000x reference)
import functools

import jax
import jax.numpy as jnp
from jax.experimental import pallas as pl
from jax.experimental.pallas import tpu as pltpu


def _block_kernel(chars_ref, lens_ref, emb_ref, wihT_ref, whhT_f_ref,
                  whhT_b_ref, bih_ref, bhh_f_ref, bhh_b_ref, wpT_ref,
                  bp_ref, ctx_ref, out_ref):
    BW, T = chars_ref.shape
    H = whhT_f_ref.shape[0]
    V = emb_ref.shape[0]

    ch = chars_ref[...]
    lens = lens_ref[...]

    iota_c = jax.lax.broadcasted_iota(jnp.int32, (BW, V), 1)
    ohs = [(ch[:, t:t + 1] == iota_c).astype(jnp.float32).astype(jnp.bfloat16)
           for t in range(T)]
    table = (jnp.dot(emb_ref[...], wihT_ref[...],
                     preferred_element_type=jnp.float32)
             + bih_ref[...]).astype(jnp.bfloat16)
    table_f = table[:, :3 * H]
    table_b = table[:, 3 * H:]

    aug_iota = jax.lax.broadcasted_iota(jnp.int32, (BW, H), 1)
    aug_ones = (aug_iota == 0).astype(jnp.float32).astype(jnp.bfloat16)

    def aug_w(whhT_ref, bhh_ref):
        return jnp.concatenate(
            [whhT_ref[...], bhh_ref[...],
             jnp.zeros((H - 1, 3 * H), jnp.float32)],
            axis=0).astype(jnp.bfloat16)

    half = jnp.bfloat16(0.5)

    def cell(xt, h, whhA):
        ha = jnp.concatenate([h, aug_ones], axis=1)
        hp = jnp.dot(ha, whhA,
                     preferred_element_type=jnp.float32).astype(jnp.bfloat16)
        rz = half * jnp.tanh(half * (xt[:, :2 * H] + hp[:, :2 * H])) + half
        r = rz[:, :H]
        z = rz[:, H:]
        n = jnp.tanh(xt[:, 2 * H:] + r * hp[:, 2 * H:])
        return n + z * (h - n)

    whhA_f = aug_w(whhT_f_ref, bhh_f_ref)
    whhA_b = aug_w(whhT_b_ref, bhh_b_ref)
    zero_b = jnp.zeros((BW, H), jnp.bfloat16)

    masks = [(t < lens).astype(jnp.float32).astype(jnp.bfloat16)
             for t in range(T)]

    h = zero_b
    outs_f = []
    for t in range(T):
        xt = jnp.dot(ohs[t], table_f,
                     preferred_element_type=jnp.float32).astype(jnp.bfloat16)
        h = cell(xt, h, whhA_f)
        outs_f.append(masks[t] * h)

    h = zero_b
    outs_b = [None] * T
    for t in range(T - 1, -1, -1):
        xt = jnp.dot(ohs[t], table_b,
                     preferred_element_type=jnp.float32).astype(jnp.bfloat16)
        hn = cell(xt, h, whhA_b)
        h = h + masks[t] * (hn - h)
        outs_b[t] = h

    wpTb = wpT_ref[...].astype(jnp.bfloat16)
    bp = bp_ref[...]
    ctxr = ctx_ref[...]
    es = []
    for t in range(T):
        oc = jnp.concatenate([outs_f[t], outs_b[t]], axis=1)
        p = jnp.tanh(jnp.dot(oc, wpTb, preferred_element_type=jnp.float32)
                     + bp)
        s_t = jnp.sum(p * ctxr, axis=1, keepdims=True)
        es.append(jnp.exp(s_t))
    den = es[0]
    for t in range(1, T):
        den = den + es[t]
    inv = 1.0 / den
    att0 = es[0] * inv
    acc_f = outs_f[0].astype(jnp.float32) * att0
    acc_b = outs_b[0].astype(jnp.float32) * att0
    for t in range(1, T):
        att_t = es[t] * inv
        acc_f = acc_f + outs_f[t].astype(jnp.float32) * att_t
        acc_b = acc_b + outs_b[t].astype(jnp.float32) * att_t
    out_ref[...] = jnp.concatenate([acc_f, acc_b], axis=1)


@functools.partial(jax.jit, static_argnames=("interpret",))
def _char_to_word(padded_char_tensor, sequence_lens, emb, Wih_f, Whh_f,
                  bih_f, bhh_f, Wih_b, Whh_b, bih_b, bhh_b, Wp, bp, ctx,
                  interpret=False):
    NW, T = padded_char_tensor.shape
    V, EMB = emb.shape
    H = Whh_f.shape[1]
    C = Wp.shape[0]
    BW = 2048 if NW % 2048 == 0 else NW
    n_blocks = NW // BW

    chars2 = padded_char_tensor.astype(jnp.int32)
    lens2 = sequence_lens.astype(jnp.int32)[:, None]
    wihT = jnp.concatenate([Wih_f.T, Wih_b.T], axis=1)
    bih = jnp.concatenate([bih_f, bih_b])[None, :]
    out = pl.pallas_call(
        _block_kernel,
        out_shape=jax.ShapeDtypeStruct((NW, 2 * H), jnp.float32),
        grid=(n_blocks,),
        in_specs=[
            pl.BlockSpec((BW, T), lambda i: (i, 0)),
            pl.BlockSpec((BW, 1), lambda i: (i, 0)),
            pl.BlockSpec((V, EMB), lambda i: (0, 0)),
            pl.BlockSpec((EMB, 6 * H), lambda i: (0, 0)),
            pl.BlockSpec((H, 3 * H), lambda i: (0, 0)),
            pl.BlockSpec((H, 3 * H), lambda i: (0, 0)),
            pl.BlockSpec((1, 6 * H), lambda i: (0, 0)),
            pl.BlockSpec((1, 3 * H), lambda i: (0, 0)),
            pl.BlockSpec((1, 3 * H), lambda i: (0, 0)),
            pl.BlockSpec((2 * H, C), lambda i: (0, 0)),
            pl.BlockSpec((1, C), lambda i: (0, 0)),
            pl.BlockSpec((1, C), lambda i: (0, 0)),
        ],
        out_specs=pl.BlockSpec((BW, 2 * H), lambda i: (i, 0)),
        compiler_params=pltpu.CompilerParams(
            dimension_semantics=("parallel",),
            vmem_limit_bytes=56 * 1024 * 1024,
        ),
        name="char_to_word",
        interpret=interpret,
    )(
        chars2, lens2, emb, wihT, Whh_f.T, Whh_b.T, bih,
        bhh_f[None, :], bhh_b[None, :], Wp.T, bp[None, :], ctx.T,
    )
    return out


def kernel(padded_char_tensor, sequence_lens, emb, Wih_f, Whh_f, bih_f,
           bhh_f, Wih_b, Whh_b, bih_b, bhh_b, Wp, bp, ctx):
    return _char_to_word(padded_char_tensor, sequence_lens, emb, Wih_f,
                         Whh_f, bih_f, bhh_f, Wih_b, Whh_b, bih_b, bhh_b,
                         Wp, bp, ctx)


# speedup vs baseline: 1.2079x; 1.2079x over previous
import functools

import jax
import jax.numpy as jnp
from jax.experimental import pallas as pl
from jax.experimental.pallas import tpu as pltpu


def _block_kernel(chars_ref, lens_ref, emb_ref, wihT_ref, whhT_f_ref,
                  whhT_b_ref, bih_ref, bhh_f_ref, bhh_b_ref, wpT_ref,
                  bp_ref, ctx_ref, out_ref):
    BW, T = chars_ref.shape
    H = whhT_f_ref.shape[0]
    V = emb_ref.shape[0]

    ch = chars_ref[...]
    lens = lens_ref[...]

    iota_c = jax.lax.broadcasted_iota(jnp.int32, (BW, V), 1)
    ohs = [(ch[:, t:t + 1] == iota_c).astype(jnp.float32).astype(jnp.bfloat16)
           for t in range(T)]
    table = (jnp.dot(emb_ref[...], wihT_ref[...],
                     preferred_element_type=jnp.float32)
             + bih_ref[...]).astype(jnp.bfloat16)
    table_f = table[:, :3 * H]
    table_b = table[:, 3 * H:]

    aug_iota = jax.lax.broadcasted_iota(jnp.int32, (BW, H), 1)
    aug_ones = (aug_iota == 0).astype(jnp.float32).astype(jnp.bfloat16)

    def aug_w(whhT_ref, bhh_ref):
        return jnp.concatenate(
            [whhT_ref[...], bhh_ref[...],
             jnp.zeros((H - 1, 3 * H), jnp.float32)],
            axis=0).astype(jnp.bfloat16)

    half = jnp.bfloat16(0.5)

    def cell(xt, h, whhA):
        ha = jnp.concatenate([h, aug_ones], axis=1)
        hp = jnp.dot(ha, whhA,
                     preferred_element_type=jnp.float32).astype(jnp.bfloat16)
        rz = half * jnp.tanh(half * (xt[:, :2 * H] + hp[:, :2 * H])) + half
        r = rz[:, :H]
        z = rz[:, H:]
        n = jnp.tanh(xt[:, 2 * H:] + r * hp[:, 2 * H:])
        return n + z * (h - n)

    whhA_f = aug_w(whhT_f_ref, bhh_f_ref)
    whhA_b = aug_w(whhT_b_ref, bhh_b_ref)
    zero_b = jnp.zeros((BW, H), jnp.bfloat16)

    masks = [(t < lens).astype(jnp.float32).astype(jnp.bfloat16)
             for t in range(T)]

    h = zero_b
    outs_f = []
    for t in range(T):
        xt = jnp.dot(ohs[t], table_f,
                     preferred_element_type=jnp.float32).astype(jnp.bfloat16)
        h = cell(xt, h, whhA_f)
        outs_f.append(masks[t] * h)

    h = zero_b
    outs_b = [None] * T
    for t in range(T - 1, -1, -1):
        xt = jnp.dot(ohs[t], table_b,
                     preferred_element_type=jnp.float32).astype(jnp.bfloat16)
        hn = cell(xt, h, whhA_b)
        h = h + masks[t] * (hn - h)
        outs_b[t] = h

    wpTb = wpT_ref[...].astype(jnp.bfloat16)
    bp = bp_ref[...]
    ctxr = ctx_ref[...]
    es = []
    for t in range(T):
        oc = jnp.concatenate([outs_f[t], outs_b[t]], axis=1)
        p = jnp.tanh(jnp.dot(oc, wpTb, preferred_element_type=jnp.float32)
                     + bp)
        s_t = jnp.sum(p * ctxr, axis=1, keepdims=True)
        es.append(jnp.exp(s_t))
    den = es[0]
    for t in range(1, T):
        den = den + es[t]
    inv = 1.0 / den
    att0 = es[0] * inv
    acc_f = outs_f[0].astype(jnp.float32) * att0
    acc_b = outs_b[0].astype(jnp.float32) * att0
    for t in range(1, T):
        att_t = es[t] * inv
        acc_f = acc_f + outs_f[t].astype(jnp.float32) * att_t
        acc_b = acc_b + outs_b[t].astype(jnp.float32) * att_t
    out_ref[...] = jnp.concatenate([acc_f, acc_b], axis=1)


@functools.partial(jax.jit, static_argnames=("interpret",))
def _char_to_word(padded_char_tensor, sequence_lens, emb, Wih_f, Whh_f,
                  bih_f, bhh_f, Wih_b, Whh_b, bih_b, bhh_b, Wp, bp, ctx,
                  interpret=False):
    NW, T = padded_char_tensor.shape
    V, EMB = emb.shape
    H = Whh_f.shape[1]
    C = Wp.shape[0]
    BW = 1024 if NW % 1024 == 0 else NW
    n_blocks = NW // BW

    chars2 = padded_char_tensor.astype(jnp.int32)
    lens2 = sequence_lens.astype(jnp.int32)[:, None]
    wihT = jnp.concatenate([Wih_f.T, Wih_b.T], axis=1)
    bih = jnp.concatenate([bih_f, bih_b])[None, :]
    out = pl.pallas_call(
        _block_kernel,
        out_shape=jax.ShapeDtypeStruct((NW, 2 * H), jnp.float32),
        grid=(n_blocks,),
        in_specs=[
            pl.BlockSpec((BW, T), lambda i: (i, 0)),
            pl.BlockSpec((BW, 1), lambda i: (i, 0)),
            pl.BlockSpec((V, EMB), lambda i: (0, 0)),
            pl.BlockSpec((EMB, 6 * H), lambda i: (0, 0)),
            pl.BlockSpec((H, 3 * H), lambda i: (0, 0)),
            pl.BlockSpec((H, 3 * H), lambda i: (0, 0)),
            pl.BlockSpec((1, 6 * H), lambda i: (0, 0)),
            pl.BlockSpec((1, 3 * H), lambda i: (0, 0)),
            pl.BlockSpec((1, 3 * H), lambda i: (0, 0)),
            pl.BlockSpec((2 * H, C), lambda i: (0, 0)),
            pl.BlockSpec((1, C), lambda i: (0, 0)),
            pl.BlockSpec((1, C), lambda i: (0, 0)),
        ],
        out_specs=pl.BlockSpec((BW, 2 * H), lambda i: (i, 0)),
        compiler_params=pltpu.CompilerParams(
            dimension_semantics=("parallel",),
            vmem_limit_bytes=56 * 1024 * 1024,
        ),
        name="char_to_word",
        interpret=interpret,
    )(
        chars2, lens2, emb, wihT, Whh_f.T, Whh_b.T, bih,
        bhh_f[None, :], bhh_b[None, :], Wp.T, bp[None, :], ctx.T,
    )
    return out


def kernel(padded_char_tensor, sequence_lens, emb, Wih_f, Whh_f, bih_f,
           bhh_f, Wih_b, Whh_b, bih_b, bhh_b, Wp, bp, ctx):
    return _char_to_word(padded_char_tensor, sequence_lens, emb, Wih_f,
                         Whh_f, bih_f, bhh_f, Wih_b, Whh_b, bih_b, bhh_b,
                         Wp, bp, ctx)
